# baseline (speedup 1.0000x reference)
import functools

import jax
import jax.numpy as jnp
from jax import lax
from jax.experimental import pallas as pl
from jax.experimental.pallas import tpu as pltpu
from jax.experimental.pallas import tpu_sc as plsc

B = 32
LATENT = 256
N_ATOMS = 1024
T = 128
KSIZE = 7
FLAT = LATENT * T
KCOL = LATENT * KSIZE
LANES = 16
NVEC = FLAT // LANES


BB = 16
NT = BB * T


def _conv_argmax_body(x_ref, w_ref, bias_ref, ai_ref, pos_ref):
    xcat = jnp.concatenate([x_ref[i] for i in range(BB)], axis=1)
    tmod = lax.bitwise_and(
        lax.broadcasted_iota(jnp.int32, (LATENT, NT), 1), T - 1
    )
    cols = [xcat]
    for k in range(1, KSIZE):
        sh = jnp.concatenate(
            [xcat[:, k:], jnp.zeros((LATENT, k), jnp.float32)], axis=1
        )
        cols.append(jnp.where(tmod < T - k, sh, 0.0))
    xcol = jnp.concatenate(cols, axis=0)
    conv = lax.dot_general(
        w_ref[...], xcol, (((1,), (0,)), ((), ())),
        preferred_element_type=jnp.float32,
    )
    conv = conv + bias_ref[...]
    ids = (
        lax.broadcasted_iota(jnp.int32, (N_ATOMS, T), 0) * T
        + lax.broadcasted_iota(jnp.int32, (N_ATOMS, T), 1)
    )
    ais, poss = [], []
    for i in range(BB):
        ci = lax.slice(conv, (0, i * T), (N_ATOMS, (i + 1) * T))
        m = jnp.max(ci)
        idx = jnp.min(jnp.where(ci == m, ids, jnp.int32(N_ATOMS * T)))
        ais.append(jnp.full((1, 1, T), idx // T, jnp.int32))
        poss.append(jnp.full((1, 1, T), idx % T, jnp.int32))
    ai_ref[...] = jnp.concatenate(ais, axis=0)
    pos_ref[...] = jnp.concatenate(poss, axis=0)


def _conv_argmax(x, wflat, bias2d):
    return pl.pallas_call(
        _conv_argmax_body,
        grid=(B // BB,),
        in_specs=[
            pl.BlockSpec((BB, LATENT, T), lambda b: (b, 0, 0)),
            pl.BlockSpec((N_ATOMS, KCOL), lambda b: (0, 0)),
            pl.BlockSpec((N_ATOMS, NT), lambda b: (0, 0)),
        ],
        out_specs=[
            pl.BlockSpec((BB, 1, T), lambda b: (b, 0, 0)),
            pl.BlockSpec((BB, 1, T), lambda b: (b, 0, 0)),
        ],
        out_shape=[
            jax.ShapeDtypeStruct((B, 1, T), jnp.int32),
            jax.ShapeDtypeStruct((B, 1, T), jnp.int32),
        ],
        compiler_params=pltpu.CompilerParams(
            dimension_semantics=("arbitrary",),
        ),
    )(x, wflat, bias2d)


def _sc_place_body(ai_hbm, pos_hbm, x_hbm, atoms_hbm, scaled_hbm, resid_hbm,
                   ai_v, pos_v, row_v, x_v, scl_v, sem):
    w = lax.axis_index("s") * 2 + lax.axis_index("c")
    VPC = T // LANES
    pltpu.sync_copy(ai_hbm.at[w], ai_v)
    rowdma = pltpu.async_copy(
        atoms_hbm.at[ai_v.at[0, pl.ds(0, 1)]], row_v, sem
    )
    pltpu.sync_copy(pos_hbm.at[w], pos_v)
    pltpu.sync_copy(x_hbm.at[w], x_v)
    rowdma.wait()

    pv = pos_v[0, pl.ds(0, LANES)]
    thr = jnp.int32(T) - pv
    iota = lax.broadcasted_iota(jnp.int32, (LANES,), 0)
    zero16 = jnp.zeros((LANES,), jnp.int32)
    mA, idxA, mB, idxB = [], [], [], []
    for j in range(VPC):
        tv = iota + (j * LANES)
        mA.append(tv < thr)
        idxA.append(jnp.minimum(tv + pv, T - 1))
        mB.append(tv >= pv)
        idxB.append(jnp.maximum(tv - pv, 0))

    zz = tuple(jnp.zeros((LANES,), jnp.float32) for _ in range(VPC))

    @plsc.parallel_loop(0, LATENT, carry=(zz, zz))
    def pass_a(c, carry):
        n2s, crs = carry
        cs = jnp.full((LANES,), c, jnp.int32)
        n2o, cro = [], []
        for j in range(VPC):
            r = row_v[0, c, pl.ds(j * LANES, LANES)]
            xg = plsc.load_gather(x_v, [cs, idxA[j]], mask=mA[j])
            rm = jnp.where(mA[j], r, 0.0)
            n2o.append(n2s[j] + rm * rm)
            cro.append(crs[j] + rm * xg)
        return tuple(n2o), tuple(cro)

    n2s, crs = pass_a
    n2v = jnp.full((LANES,), jnp.sum(sum(n2s[1:], n2s[0])))
    crv = jnp.full((LANES,), jnp.sum(sum(crs[1:], crs[0])))
    y = plsc.bitcast(
        jnp.int32(0x5F3759DF) - (plsc.bitcast(n2v, jnp.int32) >> 1),
        jnp.float32,
    )
    half = 0.5 * n2v
    for _ in range(3):
        y = y * (1.5 - half * y * y)
    norm = n2v * y
    den = (norm + 1e-8) * (norm + 1e-8)
    s = crv / den

    def run_pass_b(c0, c1):
        @plsc.parallel_loop(c0, c1)
        def _pb(c):
            cs = jnp.full((LANES,), c, jnp.int32)
            for j in range(VPC):
                rg = plsc.load_gather(
                    row_v, [zero16, cs, idxB[j]], mask=mB[j]
                )
                sc = jnp.where(mB[j], rg, 0.0) * s
                xv = x_v[c, pl.ds(j * LANES, LANES)]
                scl_v[c, pl.ds(j * LANES, LANES)] = sc
                x_v[c, pl.ds(j * LANES, LANES)] = xv - sc

    HALF = LATENT // 2
    run_pass_b(0, HALF)
    d1 = pltpu.async_copy(
        scl_v.at[pl.ds(0, HALF)], scaled_hbm.at[w, pl.ds(0, HALF)], sem
    )
    d2 = pltpu.async_copy(
        x_v.at[pl.ds(0, HALF)], resid_hbm.at[w, pl.ds(0, HALF)], sem
    )
    run_pass_b(HALF, LATENT)
    d3 = pltpu.async_copy(
        scl_v.at[pl.ds(HALF, HALF)], scaled_hbm.at[w, pl.ds(HALF, HALF)], sem
    )
    d4 = pltpu.async_copy(
        x_v.at[pl.ds(HALF, HALF)], resid_hbm.at[w, pl.ds(HALF, HALF)], sem
    )
    d1.wait()
    d2.wait()
    d3.wait()
    d4.wait()


@functools.cache
def _make_sc_place():
    return pl.kernel(
        _sc_place_body,
        out_type=(
            jax.ShapeDtypeStruct((B, LATENT, T), jnp.float32),
            jax.ShapeDtypeStruct((B, LATENT, T), jnp.float32),
        ),
        mesh=plsc.VectorSubcoreMesh(
            core_axis_name="c", subcore_axis_name="s", num_cores=2,
            num_subcores=16,
        ),
        compiler_params=pltpu.CompilerParams(
            needs_layout_passes=False, use_tc_tiling_on_sc=True,
        ),
        scratch_types=[
            pltpu.VMEM((1, T), jnp.int32),
            pltpu.VMEM((1, T), jnp.int32),
            pltpu.VMEM((1, LATENT, T), jnp.float32),
            pltpu.VMEM((LATENT, T), jnp.float32),
            pltpu.VMEM((LATENT, T), jnp.float32),
            pltpu.SemaphoreType.DMA,
        ],
    )


@jax.jit
def kernel(x, W, b, atoms):
    wflat = jnp.concatenate([W[:, :, k] for k in range(KSIZE)], axis=1)
    bias2d = jnp.broadcast_to(b[:, None], (N_ATOMS, NT))
    ai_out, pos_out = _conv_argmax(x, wflat, bias2d)
    scaled, resid = _make_sc_place()(ai_out, pos_out, x, atoms)
    return scaled, resid


# speedup vs baseline: 1.0090x; 1.0090x over previous
import functools

import jax
import jax.numpy as jnp
from jax import lax
from jax.experimental import pallas as pl
from jax.experimental.pallas import tpu as pltpu
from jax.experimental.pallas import tpu_sc as plsc

B = 32
LATENT = 256
N_ATOMS = 1024
T = 128
KSIZE = 7
FLAT = LATENT * T
KCOL = LATENT * KSIZE
LANES = 16
NVEC = FLAT // LANES


BB = 8
NT = BB * T


def _conv_argmax_body(x_ref, w_ref, bias_ref, ai_ref, pos_ref):
    xcat = jnp.concatenate([x_ref[i] for i in range(BB)], axis=1)
    tmod = lax.bitwise_and(
        lax.broadcasted_iota(jnp.int32, (LATENT, NT), 1), T - 1
    )
    cols = [xcat]
    for k in range(1, KSIZE):
        sh = jnp.concatenate(
            [xcat[:, k:], jnp.zeros((LATENT, k), jnp.float32)], axis=1
        )
        cols.append(jnp.where(tmod < T - k, sh, 0.0))
    xcol = jnp.concatenate(cols, axis=0)
    conv = lax.dot_general(
        w_ref[...], xcol, (((1,), (0,)), ((), ())),
        preferred_element_type=jnp.float32,
    )
    conv = conv + bias_ref[...]
    ids = (
        lax.broadcasted_iota(jnp.int32, (N_ATOMS, T), 0) * T
        + lax.broadcasted_iota(jnp.int32, (N_ATOMS, T), 1)
    )
    ais, poss = [], []
    for i in range(BB):
        ci = lax.slice(conv, (0, i * T), (N_ATOMS, (i + 1) * T))
        m = jnp.max(ci)
        idx = jnp.min(jnp.where(ci == m, ids, jnp.int32(N_ATOMS * T)))
        ais.append(jnp.full((1, 1, T), idx // T, jnp.int32))
        poss.append(jnp.full((1, 1, T), idx % T, jnp.int32))
    ai_ref[...] = jnp.concatenate(ais, axis=0)
    pos_ref[...] = jnp.concatenate(poss, axis=0)


def _conv_argmax(x, wflat, bias2d):
    return pl.pallas_call(
        _conv_argmax_body,
        grid=(B // BB,),
        in_specs=[
            pl.BlockSpec((BB, LATENT, T), lambda b: (b, 0, 0)),
            pl.BlockSpec((N_ATOMS, KCOL), lambda b: (0, 0)),
            pl.BlockSpec((N_ATOMS, NT), lambda b: (0, 0)),
        ],
        out_specs=[
            pl.BlockSpec((BB, 1, T), lambda b: (b, 0, 0)),
            pl.BlockSpec((BB, 1, T), lambda b: (b, 0, 0)),
        ],
        out_shape=[
            jax.ShapeDtypeStruct((B, 1, T), jnp.int32),
            jax.ShapeDtypeStruct((B, 1, T), jnp.int32),
        ],
        compiler_params=pltpu.CompilerParams(
            dimension_semantics=("parallel",),
        ),
    )(x, wflat, bias2d)


def _sc_place_body(ai_hbm, pos_hbm, x_hbm, atoms_hbm, scaled_hbm, resid_hbm,
                   ai_v, pos_v, row_v, x_v, scl_v, sem):
    w = lax.axis_index("s") * 2 + lax.axis_index("c")
    VPC = T // LANES
    pltpu.sync_copy(ai_hbm.at[w], ai_v)
    rowdma = pltpu.async_copy(
        atoms_hbm.at[ai_v.at[0, pl.ds(0, 1)]], row_v, sem
    )
    pltpu.sync_copy(pos_hbm.at[w], pos_v)
    pltpu.sync_copy(x_hbm.at[w], x_v)
    rowdma.wait()

    pv = pos_v[0, pl.ds(0, LANES)]
    thr = jnp.int32(T) - pv
    iota = lax.broadcasted_iota(jnp.int32, (LANES,), 0)
    zero16 = jnp.zeros((LANES,), jnp.int32)
    mA, idxA, mB, idxB = [], [], [], []
    for j in range(VPC):
        tv = iota + (j * LANES)
        mA.append(tv < thr)
        idxA.append(jnp.minimum(tv + pv, T - 1))
        mB.append(tv >= pv)
        idxB.append(jnp.maximum(tv - pv, 0))

    zz = tuple(jnp.zeros((LANES,), jnp.float32) for _ in range(VPC))

    @plsc.parallel_loop(0, LATENT, carry=(zz, zz))
    def pass_a(c, carry):
        n2s, crs = carry
        cs = jnp.full((LANES,), c, jnp.int32)
        n2o, cro = [], []
        for j in range(VPC):
            r = row_v[0, c, pl.ds(j * LANES, LANES)]
            xg = plsc.load_gather(x_v, [cs, idxA[j]], mask=mA[j])
            rm = jnp.where(mA[j], r, 0.0)
            n2o.append(n2s[j] + rm * rm)
            cro.append(crs[j] + rm * xg)
        return tuple(n2o), tuple(cro)

    n2s, crs = pass_a
    n2v = jnp.full((LANES,), jnp.sum(sum(n2s[1:], n2s[0])))
    crv = jnp.full((LANES,), jnp.sum(sum(crs[1:], crs[0])))
    y = plsc.bitcast(
        jnp.int32(0x5F3759DF) - (plsc.bitcast(n2v, jnp.int32) >> 1),
        jnp.float32,
    )
    half = 0.5 * n2v
    for _ in range(3):
        y = y * (1.5 - half * y * y)
    norm = n2v * y
    den = (norm + 1e-8) * (norm + 1e-8)
    s = crv / den

    def run_pass_b(c0, c1):
        @plsc.parallel_loop(c0, c1)
        def _pb(c):
            cs = jnp.full((LANES,), c, jnp.int32)
            for j in range(VPC):
                rg = plsc.load_gather(
                    row_v, [zero16, cs, idxB[j]], mask=mB[j]
                )
                sc = jnp.where(mB[j], rg, 0.0) * s
                xv = x_v[c, pl.ds(j * LANES, LANES)]
                scl_v[c, pl.ds(j * LANES, LANES)] = sc
                x_v[c, pl.ds(j * LANES, LANES)] = xv - sc

    HALF = LATENT // 2
    run_pass_b(0, HALF)
    d1 = pltpu.async_copy(
        scl_v.at[pl.ds(0, HALF)], scaled_hbm.at[w, pl.ds(0, HALF)], sem
    )
    d2 = pltpu.async_copy(
        x_v.at[pl.ds(0, HALF)], resid_hbm.at[w, pl.ds(0, HALF)], sem
    )
    run_pass_b(HALF, LATENT)
    d3 = pltpu.async_copy(
        scl_v.at[pl.ds(HALF, HALF)], scaled_hbm.at[w, pl.ds(HALF, HALF)], sem
    )
    d4 = pltpu.async_copy(
        x_v.at[pl.ds(HALF, HALF)], resid_hbm.at[w, pl.ds(HALF, HALF)], sem
    )
    d1.wait()
    d2.wait()
    d3.wait()
    d4.wait()


@functools.cache
def _make_sc_place():
    return pl.kernel(
        _sc_place_body,
        out_type=(
            jax.ShapeDtypeStruct((B, LATENT, T), jnp.float32),
            jax.ShapeDtypeStruct((B, LATENT, T), jnp.float32),
        ),
        mesh=plsc.VectorSubcoreMesh(
            core_axis_name="c", subcore_axis_name="s", num_cores=2,
            num_subcores=16,
        ),
        compiler_params=pltpu.CompilerParams(
            needs_layout_passes=False, use_tc_tiling_on_sc=True,
        ),
        scratch_types=[
            pltpu.VMEM((1, T), jnp.int32),
            pltpu.VMEM((1, T), jnp.int32),
            pltpu.VMEM((1, LATENT, T), jnp.float32),
            pltpu.VMEM((LATENT, T), jnp.float32),
            pltpu.VMEM((LATENT, T), jnp.float32),
            pltpu.SemaphoreType.DMA,
        ],
    )


@jax.jit
def kernel(x, W, b, atoms):
    wflat = jnp.concatenate([W[:, :, k] for k in range(KSIZE)], axis=1)
    bias2d = jnp.broadcast_to(b[:, None], (N_ATOMS, NT))
    ai_out, pos_out = _conv_argmax(x, wflat, bias2d)
    scaled, resid = _make_sc_place()(ai_out, pos_out, x, atoms)
    return scaled, resid


# speedup vs baseline: 1.0146x; 1.0056x over previous
import functools

import jax
import jax.numpy as jnp
from jax import lax
from jax.experimental import pallas as pl
from jax.experimental.pallas import tpu as pltpu
from jax.experimental.pallas import tpu_sc as plsc

B = 32
LATENT = 256
N_ATOMS = 1024
T = 128
KSIZE = 7
FLAT = LATENT * T
KCOL = LATENT * KSIZE
LANES = 16
NVEC = FLAT // LANES


BB = 8
NT = BB * T


def _conv_argmax_body(x_ref, w_ref, bias_ref, ip_ref):
    xcat = jnp.concatenate([x_ref[i] for i in range(BB)], axis=1)
    tmod = lax.bitwise_and(
        lax.broadcasted_iota(jnp.int32, (LATENT, NT), 1), T - 1
    )
    cols = [xcat]
    for k in range(1, KSIZE):
        sh = jnp.concatenate(
            [xcat[:, k:], jnp.zeros((LATENT, k), jnp.float32)], axis=1
        )
        cols.append(jnp.where(tmod < T - k, sh, 0.0))
    xcol = jnp.concatenate(cols, axis=0)
    conv = lax.dot_general(
        w_ref[...], xcol, (((1,), (0,)), ((), ())),
        preferred_element_type=jnp.float32,
    )
    conv = conv + bias_ref[...]
    ids = (
        lax.broadcasted_iota(jnp.int32, (N_ATOMS, T), 0) * T
        + lax.broadcasted_iota(jnp.int32, (N_ATOMS, T), 1)
    )
    outs = []
    for i in range(BB):
        ci = lax.slice(conv, (0, i * T), (N_ATOMS, (i + 1) * T))
        m = jnp.max(ci)
        idx = jnp.min(jnp.where(ci == m, ids, jnp.int32(N_ATOMS * T)))
        outs.append(
            jnp.concatenate(
                [jnp.full((1, 1, T), idx // T, jnp.int32),
                 jnp.full((1, 1, T), idx % T, jnp.int32)],
                axis=1,
            )
        )
    ip_ref[...] = jnp.concatenate(outs, axis=0)


def _conv_argmax(x, wflat, bias2d):
    return pl.pallas_call(
        _conv_argmax_body,
        grid=(B // BB,),
        in_specs=[
            pl.BlockSpec((BB, LATENT, T), lambda b: (b, 0, 0)),
            pl.BlockSpec((N_ATOMS, KCOL), lambda b: (0, 0)),
            pl.BlockSpec((N_ATOMS, NT), lambda b: (0, 0)),
        ],
        out_specs=pl.BlockSpec((BB, 2, T), lambda b: (b, 0, 0)),
        out_shape=jax.ShapeDtypeStruct((B, 2, T), jnp.int32),
        compiler_params=pltpu.CompilerParams(
            dimension_semantics=("parallel",),
        ),
    )(x, wflat, bias2d)


def _sc_place_body(ip_hbm, x_hbm, atoms_hbm, scaled_hbm, resid_hbm,
                   ip_v, row_v, x_v, scl_v, sem, xsem):
    w = lax.axis_index("s") * 2 + lax.axis_index("c")
    VPC = T // LANES
    xdma = pltpu.async_copy(x_hbm.at[w], x_v, xsem)
    pltpu.sync_copy(ip_hbm.at[w], ip_v)
    rowdma = pltpu.async_copy(
        atoms_hbm.at[ip_v.at[0, pl.ds(0, 1)]], row_v, sem
    )
    xdma.wait()
    rowdma.wait()

    pv = ip_v[1, pl.ds(0, LANES)]
    thr = jnp.int32(T) - pv
    iota = lax.broadcasted_iota(jnp.int32, (LANES,), 0)
    zero16 = jnp.zeros((LANES,), jnp.int32)
    mA, idxA, mB, idxB = [], [], [], []
    for j in range(VPC):
        tv = iota + (j * LANES)
        mA.append(tv < thr)
        idxA.append(jnp.minimum(tv + pv, T - 1))
        mB.append(tv >= pv)
        idxB.append(jnp.maximum(tv - pv, 0))

    zz = tuple(jnp.zeros((LANES,), jnp.float32) for _ in range(VPC))

    @plsc.parallel_loop(0, LATENT, carry=(zz, zz))
    def pass_a(c, carry):
        n2s, crs = carry
        cs = jnp.full((LANES,), c, jnp.int32)
        n2o, cro = [], []
        for j in range(VPC):
            r = row_v[0, c, pl.ds(j * LANES, LANES)]
            xg = plsc.load_gather(x_v, [cs, idxA[j]], mask=mA[j])
            rm = jnp.where(mA[j], r, 0.0)
            n2o.append(n2s[j] + rm * rm)
            cro.append(crs[j] + rm * xg)
        return tuple(n2o), tuple(cro)

    n2s, crs = pass_a
    n2v = jnp.full((LANES,), jnp.sum(sum(n2s[1:], n2s[0])))
    crv = jnp.full((LANES,), jnp.sum(sum(crs[1:], crs[0])))
    y = plsc.bitcast(
        jnp.int32(0x5F3759DF) - (plsc.bitcast(n2v, jnp.int32) >> 1),
        jnp.float32,
    )
    half = 0.5 * n2v
    for _ in range(3):
        y = y * (1.5 - half * y * y)
    norm = n2v * y
    den = (norm + 1e-8) * (norm + 1e-8)
    s = crv / den

    def run_pass_b(c0, c1):
        @plsc.parallel_loop(c0, c1)
        def _pb(c):
            cs = jnp.full((LANES,), c, jnp.int32)
            for j in range(VPC):
                rg = plsc.load_gather(
                    row_v, [zero16, cs, idxB[j]], mask=mB[j]
                )
                sc = jnp.where(mB[j], rg, 0.0) * s
                xv = x_v[c, pl.ds(j * LANES, LANES)]
                scl_v[c, pl.ds(j * LANES, LANES)] = sc
                x_v[c, pl.ds(j * LANES, LANES)] = xv - sc

    HALF = LATENT // 2
    run_pass_b(0, HALF)
    d1 = pltpu.async_copy(
        scl_v.at[pl.ds(0, HALF)], scaled_hbm.at[w, pl.ds(0, HALF)], sem
    )
    d2 = pltpu.async_copy(
        x_v.at[pl.ds(0, HALF)], resid_hbm.at[w, pl.ds(0, HALF)], sem
    )
    run_pass_b(HALF, LATENT)
    d3 = pltpu.async_copy(
        scl_v.at[pl.ds(HALF, HALF)], scaled_hbm.at[w, pl.ds(HALF, HALF)], sem
    )
    d4 = pltpu.async_copy(
        x_v.at[pl.ds(HALF, HALF)], resid_hbm.at[w, pl.ds(HALF, HALF)], sem
    )
    d1.wait()
    d2.wait()
    d3.wait()
    d4.wait()


@functools.cache
def _make_sc_place():
    return pl.kernel(
        _sc_place_body,
        out_type=(
            jax.ShapeDtypeStruct((B, LATENT, T), jnp.float32),
            jax.ShapeDtypeStruct((B, LATENT, T), jnp.float32),
        ),
        mesh=plsc.VectorSubcoreMesh(
            core_axis_name="c", subcore_axis_name="s", num_cores=2,
            num_subcores=16,
        ),
        compiler_params=pltpu.CompilerParams(
            needs_layout_passes=False, use_tc_tiling_on_sc=True,
        ),
        scratch_types=[
            pltpu.VMEM((2, T), jnp.int32),
            pltpu.VMEM((1, LATENT, T), jnp.float32),
            pltpu.VMEM((LATENT, T), jnp.float32),
            pltpu.VMEM((LATENT, T), jnp.float32),
            pltpu.SemaphoreType.DMA,
            pltpu.SemaphoreType.DMA,
        ],
    )


@jax.jit
def kernel(x, W, b, atoms):
    wflat = jnp.concatenate([W[:, :, k] for k in range(KSIZE)], axis=1)
    bias2d = jnp.broadcast_to(b[:, None], (N_ATOMS, NT))
    ip_out = _conv_argmax(x, wflat, bias2d)
    scaled, resid = _make_sc_place()(ip_out, x, atoms)
    return scaled, resid
